# Initial kernel scaffold; baseline (speedup 1.0000x reference)
import jax
import jax.numpy as jnp
from jax.experimental import pallas as pl


def kernel(rpn_features, anchors, img_sz, W_cls, b_cls, W_bbox, b_bbox):
    raise NotImplementedError("write your pallas kernel here")


# speedup vs baseline: 14.0201x; 14.0201x over previous
import jax
import jax.numpy as jnp
from jax.experimental import pallas as pl
from jax.experimental.pallas import tpu as pltpu

A = 9
H = 50
W = 50
C = 512
N = H * W * A
NPIX = H * W
MAX_OUT = 300
IOU_THR = 0.7
NEG = -1e10
ROWS = 176
LANES = 128
NPAD = ROWS * LANES
BIG = 1e9


def _matmul_kernel(f_ref, w_ref, b_ref, o_ref):
    o_ref[...] = (
        jnp.dot(f_ref[...], w_ref[...], preferred_element_type=jnp.float32)
        + b_ref[...]
    )


def _nms_kernel(img_ref, s1_ref, s0_ref, tx_ref, ty_ref, tw_ref, th_ref,
                a0_ref, a1_ref, a2_ref, a3_ref, prop_ref, score_ref):
    lin = (
        jax.lax.broadcasted_iota(jnp.int32, (ROWS, LANES), 0) * LANES
        + jax.lax.broadcasted_iota(jnp.int32, (ROWS, LANES), 1)
    )
    is_real = lin < N

    s1 = s1_ref[...]
    s0 = s0_ref[...]
    mx = jnp.maximum(s0, s1)
    e1 = jnp.exp(s1 - mx)
    e0 = jnp.exp(s0 - mx)
    prob = e1 / (e0 + e1)
    scores0 = jnp.where(is_real, prob, NEG)

    a0 = a0_ref[...]
    a1 = a1_ref[...]
    a2 = a2_ref[...]
    a3 = a3_ref[...]
    w = a3 - a1 + 1.0
    h = a2 - a0 + 1.0
    x = a0 + 0.5 * h
    y = a1 + 0.5 * w
    x_pred = tx_ref[...] * h + x
    y_pred = ty_ref[...] * w + y
    w_pred = jnp.exp(tw_ref[...]) * w
    h_pred = jnp.exp(th_ref[...]) * h
    img_w = img_ref[0]
    img_h = img_ref[1]
    bx0 = jnp.maximum(x_pred - 0.5 * h_pred, 0.0)
    bx1 = jnp.minimum(x_pred + 0.5 * h_pred, img_w)
    by0 = jnp.maximum(y_pred - 0.5 * w_pred, 0.0)
    by1 = jnp.minimum(y_pred + 0.5 * w_pred, img_h)
    areas = jnp.maximum(bx1 - bx0, 0.0) * jnp.maximum(by1 - by0, 0.0)

    col4 = jax.lax.broadcasted_iota(jnp.int32, (1, 4), 1)

    def step(t, valid):
        m = jnp.max(valid)
        idx = jnp.min(jnp.where(valid == m, lin, jnp.int32(NPAD)))
        onehot = lin == idx
        sx0 = jnp.sum(jnp.where(onehot, bx0, 0.0))
        sy0 = jnp.sum(jnp.where(onehot, by0, 0.0))
        sx1 = jnp.sum(jnp.where(onehot, bx1, 0.0))
        sy1 = jnp.sum(jnp.where(onehot, by1, 0.0))
        barea = jnp.maximum(sx1 - sx0, 0.0) * jnp.maximum(sy1 - sy0, 0.0)
        inter = (
            jnp.maximum(jnp.minimum(bx1, sx1) - jnp.maximum(bx0, sx0), 0.0)
            * jnp.maximum(jnp.minimum(by1, sy1) - jnp.maximum(by0, sy0), 0.0)
        )
        iou = inter / (areas + barea - inter + 1e-8)
        maskf = jnp.where(m > NEG * 0.5, 1.0, 0.0)
        row = jnp.where(
            col4 == 0, sx0,
            jnp.where(col4 == 1, sy0, jnp.where(col4 == 2, sx1, sy1)),
        )
        prop_ref[pl.ds(t, 1), :] = row * maskf
        score_ref[pl.ds(t, 1), :] = jnp.full((1, 1), m * maskf, jnp.float32)
        return jnp.where(iou > IOU_THR, NEG, valid)

    jax.lax.fori_loop(0, MAX_OUT, step, scores0)


def _to_grid(v):
    return jnp.pad(v, (0, NPAD - N)).reshape(ROWS, LANES)


@jax.jit
def kernel(rpn_features, anchors, img_sz, W_cls, b_cls, W_bbox, b_bbox):
    f2d = rpn_features.reshape(NPIX, C)
    fg = jnp.arange(A, 2 * A)
    partner = fg ^ 1
    w_all = jnp.concatenate(
        [W_cls[:, fg], W_cls[:, partner],
         W_bbox[:, 0::4], W_bbox[:, 1::4], W_bbox[:, 2::4], W_bbox[:, 3::4]],
        axis=1,
    )
    b_all = jnp.concatenate(
        [b_cls[fg], b_cls[partner],
         b_bbox[0::4], b_bbox[1::4], b_bbox[2::4], b_bbox[3::4]]
    ).reshape(1, 6 * A)

    m = pl.pallas_call(
        _matmul_kernel,
        out_shape=jax.ShapeDtypeStruct((NPIX, 6 * A), jnp.float32),
    )(f2d, w_all, b_all)

    groups = [_to_grid(m[:, i * A:(i + 1) * A].reshape(-1)) for i in range(6)]
    anc = [_to_grid(anchors[:, i]) for i in range(4)]

    props, scores = pl.pallas_call(
        _nms_kernel,
        in_specs=[pl.BlockSpec(memory_space=pltpu.SMEM)]
        + [pl.BlockSpec((ROWS, LANES), lambda: (0, 0))] * 10,
        out_specs=[
            pl.BlockSpec((MAX_OUT, 4), lambda: (0, 0)),
            pl.BlockSpec((MAX_OUT, 1), lambda: (0, 0)),
        ],
        out_shape=[
            jax.ShapeDtypeStruct((MAX_OUT, 4), jnp.float32),
            jax.ShapeDtypeStruct((MAX_OUT, 1), jnp.float32),
        ],
    )(img_sz, *groups, *anc)

    return props, scores.reshape(MAX_OUT)


# speedup vs baseline: 14.5513x; 1.0379x over previous
import jax
import jax.numpy as jnp
from jax.experimental import pallas as pl
from jax.experimental.pallas import tpu as pltpu

A = 9
H = 50
W = 50
C = 512
N = H * W * A
NPIX = H * W
MAX_OUT = 300
IOU_THR = 0.7
NEG = -1e10
ROWS = 176
LANES = 128
NPAD = ROWS * LANES
BIG = 1e9


def _matmul_kernel(f_ref, w_ref, b_ref, o_ref):
    o_ref[...] = (
        jnp.dot(f_ref[...], w_ref[...], preferred_element_type=jnp.float32)
        + b_ref[...]
    )


def _nms_kernel(img_ref, s1_ref, s0_ref, tx_ref, ty_ref, tw_ref, th_ref,
                a0_ref, a1_ref, a2_ref, a3_ref, prop_ref, score_ref,
                cx0_ref, cy0_ref, cx1_ref, cy1_ref):
    lin = (
        jax.lax.broadcasted_iota(jnp.int32, (ROWS, LANES), 0) * LANES
        + jax.lax.broadcasted_iota(jnp.int32, (ROWS, LANES), 1)
    )
    is_real = lin < N

    s1 = s1_ref[...]
    s0 = s0_ref[...]
    mx = jnp.maximum(s0, s1)
    e1 = jnp.exp(s1 - mx)
    e0 = jnp.exp(s0 - mx)
    prob = e1 / (e0 + e1)
    scores0 = jnp.where(is_real, prob, NEG)

    a0 = a0_ref[...]
    a1 = a1_ref[...]
    a2 = a2_ref[...]
    a3 = a3_ref[...]
    w = a3 - a1 + 1.0
    h = a2 - a0 + 1.0
    x = a0 + 0.5 * h
    y = a1 + 0.5 * w
    x_pred = tx_ref[...] * h + x
    y_pred = ty_ref[...] * w + y
    w_pred = jnp.exp(tw_ref[...]) * w
    h_pred = jnp.exp(th_ref[...]) * h
    img_w = img_ref[0]
    img_h = img_ref[1]
    bx0 = jnp.maximum(x_pred - 0.5 * h_pred, 0.0)
    bx1 = jnp.minimum(x_pred + 0.5 * h_pred, img_w)
    by0 = jnp.maximum(y_pred - 0.5 * w_pred, 0.0)
    by1 = jnp.minimum(y_pred + 0.5 * w_pred, img_h)
    areas = jnp.maximum(bx1 - bx0, 0.0) * jnp.maximum(by1 - by0, 0.0)

    cx0_ref[...] = bx0
    cy0_ref[...] = by0
    cx1_ref[...] = bx1
    cy1_ref[...] = by1

    col4 = jax.lax.broadcasted_iota(jnp.int32, (1, 4), 1)
    lane = jax.lax.broadcasted_iota(jnp.int32, (1, LANES), 1)

    m0 = jnp.max(scores0)
    idx0 = jnp.min(jnp.where(scores0 == m0, lin, jnp.int32(NPAD)))

    def step(t, carry):
        valid, m, idx = carry
        r = idx // LANES
        onerow = lane == (idx - r * LANES)
        sx0 = jnp.sum(jnp.where(onerow, cx0_ref[pl.ds(r, 1), :], 0.0))
        sy0 = jnp.sum(jnp.where(onerow, cy0_ref[pl.ds(r, 1), :], 0.0))
        sx1 = jnp.sum(jnp.where(onerow, cx1_ref[pl.ds(r, 1), :], 0.0))
        sy1 = jnp.sum(jnp.where(onerow, cy1_ref[pl.ds(r, 1), :], 0.0))
        barea = jnp.maximum(sx1 - sx0, 0.0) * jnp.maximum(sy1 - sy0, 0.0)
        inter = (
            jnp.maximum(jnp.minimum(bx1, sx1) - jnp.maximum(bx0, sx0), 0.0)
            * jnp.maximum(jnp.minimum(by1, sy1) - jnp.maximum(by0, sy0), 0.0)
        )
        iou = inter / (areas + barea - inter + 1e-8)
        maskf = jnp.where(m > NEG * 0.5, 1.0, 0.0)
        row = jnp.where(
            col4 == 0, sx0,
            jnp.where(col4 == 1, sy0, jnp.where(col4 == 2, sx1, sy1)),
        )
        prop_ref[pl.ds(t, 1), :] = row * maskf
        score_ref[pl.ds(t, 1), :] = jnp.full((1, 1), m * maskf, jnp.float32)
        new_valid = jnp.where(iou > IOU_THR, NEG, valid)
        m2 = jnp.max(new_valid)
        idx2 = jnp.min(jnp.where(new_valid == m2, lin, jnp.int32(NPAD)))
        return new_valid, m2, idx2

    jax.lax.fori_loop(0, MAX_OUT, step, (scores0, m0, idx0))


def _to_grid(v):
    return jnp.pad(v, (0, NPAD - N)).reshape(ROWS, LANES)


@jax.jit
def kernel(rpn_features, anchors, img_sz, W_cls, b_cls, W_bbox, b_bbox):
    f2d = rpn_features.reshape(NPIX, C)
    fg = jnp.arange(A, 2 * A)
    partner = fg ^ 1
    w_all = jnp.concatenate(
        [W_cls[:, fg], W_cls[:, partner],
         W_bbox[:, 0::4], W_bbox[:, 1::4], W_bbox[:, 2::4], W_bbox[:, 3::4]],
        axis=1,
    )
    b_all = jnp.concatenate(
        [b_cls[fg], b_cls[partner],
         b_bbox[0::4], b_bbox[1::4], b_bbox[2::4], b_bbox[3::4]]
    ).reshape(1, 6 * A)

    m = pl.pallas_call(
        _matmul_kernel,
        out_shape=jax.ShapeDtypeStruct((NPIX, 6 * A), jnp.float32),
    )(f2d, w_all, b_all)

    groups = [_to_grid(m[:, i * A:(i + 1) * A].reshape(-1)) for i in range(6)]
    anc = [_to_grid(anchors[:, i]) for i in range(4)]

    props, scores = pl.pallas_call(
        _nms_kernel,
        in_specs=[pl.BlockSpec(memory_space=pltpu.SMEM)]
        + [pl.BlockSpec((ROWS, LANES), lambda: (0, 0))] * 10,
        out_specs=[
            pl.BlockSpec((MAX_OUT, 4), lambda: (0, 0)),
            pl.BlockSpec((MAX_OUT, 1), lambda: (0, 0)),
        ],
        out_shape=[
            jax.ShapeDtypeStruct((MAX_OUT, 4), jnp.float32),
            jax.ShapeDtypeStruct((MAX_OUT, 1), jnp.float32),
        ],
        scratch_shapes=[pltpu.VMEM((ROWS, LANES), jnp.float32)] * 4,
    )(img_sz, *groups, *anc)

    return props, scores.reshape(MAX_OUT)


# speedup vs baseline: 18.8227x; 1.2935x over previous
import jax
import jax.numpy as jnp
from jax.experimental import pallas as pl
from jax.experimental.pallas import tpu as pltpu

A = 9
H = 50
W = 50
C = 512
N = H * W * A
NPIX = H * W
MAX_OUT = 300
IOU_THR = 0.7
NEG = -1e10
ROWS = 176
LANES = 128
NPAD = ROWS * LANES
BIG = 1e9


def _matmul_kernel(f_ref, w_ref, b_ref, o_ref):
    o_ref[...] = (
        jnp.dot(f_ref[...], w_ref[...], preferred_element_type=jnp.float32)
        + b_ref[...]
    )


CH = 8
NCH = ROWS // CH


def _tree(xs, op):
    while len(xs) > 1:
        ys = [op(xs[i], xs[i + 1]) for i in range(0, len(xs) - 1, 2)]
        if len(xs) % 2:
            ys.append(xs[-1])
        xs = ys
    return xs[0]


def _nms_kernel(img_ref, s1_ref, s0_ref, tx_ref, ty_ref, tw_ref, th_ref,
                a0_ref, a1_ref, a2_ref, a3_ref, prop_ref, score_ref,
                v_ref, x0_ref, y0_ref, x1_ref, y1_ref, ar_ref):
    lin = (
        jax.lax.broadcasted_iota(jnp.int32, (ROWS, LANES), 0) * LANES
        + jax.lax.broadcasted_iota(jnp.int32, (ROWS, LANES), 1)
    )
    is_real = lin < N

    s1 = s1_ref[...]
    s0 = s0_ref[...]
    mx = jnp.maximum(s0, s1)
    e1 = jnp.exp(s1 - mx)
    e0 = jnp.exp(s0 - mx)
    prob = e1 / (e0 + e1)
    scores0 = jnp.where(is_real, prob, NEG)

    a0 = a0_ref[...]
    a1 = a1_ref[...]
    a2 = a2_ref[...]
    a3 = a3_ref[...]
    w = a3 - a1 + 1.0
    h = a2 - a0 + 1.0
    x = a0 + 0.5 * h
    y = a1 + 0.5 * w
    x_pred = tx_ref[...] * h + x
    y_pred = ty_ref[...] * w + y
    w_pred = jnp.exp(tw_ref[...]) * w
    h_pred = jnp.exp(th_ref[...]) * h
    img_w = img_ref[0]
    img_h = img_ref[1]
    bx0 = jnp.maximum(x_pred - 0.5 * h_pred, 0.0)
    bx1 = jnp.minimum(x_pred + 0.5 * h_pred, img_w)
    by0 = jnp.maximum(y_pred - 0.5 * w_pred, 0.0)
    by1 = jnp.minimum(y_pred + 0.5 * w_pred, img_h)
    areas = jnp.maximum(bx1 - bx0, 0.0) * jnp.maximum(by1 - by0, 0.0)

    v_ref[...] = scores0
    x0_ref[...] = bx0
    y0_ref[...] = by0
    x1_ref[...] = bx1
    y1_ref[...] = by1
    ar_ref[...] = areas

    col4 = jax.lax.broadcasted_iota(jnp.int32, (1, 4), 1)
    iota8 = (
        jax.lax.broadcasted_iota(jnp.int32, (CH, LANES), 0) * LANES
        + jax.lax.broadcasted_iota(jnp.int32, (CH, LANES), 1)
    )
    def _chunk(c):
        return pl.ds(c * CH, CH)

    def _select():
        m = jnp.max(_tree([v_ref[_chunk(c), :] for c in range(NCH)],
                          jnp.maximum))
        ohs = [v_ref[_chunk(c), :] == m for c in range(NCH)]
        cnt = jnp.sum(_tree([jnp.where(oh, 1.0, 0.0) for oh in ohs],
                            jnp.add))
        quick = [
            jnp.sum(_tree([jnp.where(ohs[c], ref[_chunk(c), :], 0.0)
                           for c in range(NCH)], jnp.add))
            for ref in (x0_ref, y0_ref, x1_ref, y1_ref)
        ]

        def exact(_):
            idx = jnp.min(_tree(
                [jnp.where(v_ref[_chunk(c), :] == m,
                           iota8 + c * CH * LANES, jnp.int32(NPAD))
                 for c in range(NCH)],
                jnp.minimum,
            ))
            return tuple(
                jnp.sum(_tree(
                    [jnp.where(iota8 + c * CH * LANES == idx,
                               ref[_chunk(c), :], 0.0)
                     for c in range(NCH)], jnp.add))
                for ref in (x0_ref, y0_ref, x1_ref, y1_ref)
            )

        rare = jnp.logical_and(cnt != 1.0, m > NEG * 0.5)
        sx0, sy0, sx1, sy1 = jax.lax.cond(
            rare, exact, lambda _: tuple(quick), None)
        return m, sx0, sy0, sx1, sy1

    def step(t, carry):
        m, sx0, sy0, sx1, sy1 = carry
        maskf = jnp.where(m > NEG * 0.5, 1.0, 0.0)
        row = jnp.where(
            col4 == 0, sx0,
            jnp.where(col4 == 1, sy0, jnp.where(col4 == 2, sx1, sy1)),
        )
        prop_ref[pl.ds(t, 1), :] = row * maskf
        score_ref[pl.ds(t, 1), :] = jnp.full((1, 1), m * maskf, jnp.float32)
        barea = jnp.maximum(sx1 - sx0, 0.0) * jnp.maximum(sy1 - sy0, 0.0)
        for c in range(NCH):
            sl = _chunk(c)
            inter = (
                jnp.maximum(jnp.minimum(x1_ref[sl, :], sx1)
                            - jnp.maximum(x0_ref[sl, :], sx0), 0.0)
                * jnp.maximum(jnp.minimum(y1_ref[sl, :], sy1)
                              - jnp.maximum(y0_ref[sl, :], sy0), 0.0)
            )
            iou = inter / (ar_ref[sl, :] + barea - inter + 1e-8)
            v_ref[sl, :] = jnp.where(iou > IOU_THR, NEG, v_ref[sl, :])
        return _select()

    jax.lax.fori_loop(0, MAX_OUT, step, _select())


def _to_grid(v):
    return jnp.pad(v, (0, NPAD - N)).reshape(ROWS, LANES)


@jax.jit
def kernel(rpn_features, anchors, img_sz, W_cls, b_cls, W_bbox, b_bbox):
    f2d = rpn_features.reshape(NPIX, C)
    fg = jnp.arange(A, 2 * A)
    partner = fg ^ 1
    w_all = jnp.concatenate(
        [W_cls[:, fg], W_cls[:, partner],
         W_bbox[:, 0::4], W_bbox[:, 1::4], W_bbox[:, 2::4], W_bbox[:, 3::4]],
        axis=1,
    )
    b_all = jnp.concatenate(
        [b_cls[fg], b_cls[partner],
         b_bbox[0::4], b_bbox[1::4], b_bbox[2::4], b_bbox[3::4]]
    ).reshape(1, 6 * A)

    m = pl.pallas_call(
        _matmul_kernel,
        out_shape=jax.ShapeDtypeStruct((NPIX, 6 * A), jnp.float32),
    )(f2d, w_all, b_all)

    groups = [_to_grid(m[:, i * A:(i + 1) * A].reshape(-1)) for i in range(6)]
    anc = [_to_grid(anchors[:, i]) for i in range(4)]

    props, scores = pl.pallas_call(
        _nms_kernel,
        in_specs=[pl.BlockSpec(memory_space=pltpu.SMEM)]
        + [pl.BlockSpec((ROWS, LANES), lambda: (0, 0))] * 10,
        out_specs=[
            pl.BlockSpec((MAX_OUT, 4), lambda: (0, 0)),
            pl.BlockSpec((MAX_OUT, 1), lambda: (0, 0)),
        ],
        out_shape=[
            jax.ShapeDtypeStruct((MAX_OUT, 4), jnp.float32),
            jax.ShapeDtypeStruct((MAX_OUT, 1), jnp.float32),
        ],
        scratch_shapes=[pltpu.VMEM((ROWS, LANES), jnp.float32)] * 6,
    )(img_sz, *groups, *anc)

    return props, scores.reshape(MAX_OUT)
